# baseline (device time: 39881 ns/iter reference)
import jax
import jax.numpy as jnp
from jax import lax
from jax.experimental import pallas as pl
from jax.experimental.pallas import tpu as pltpu

N_DEV = 8
MASKS = (1, 3, 4)

GROUPS = tuple(
    (col, 128, rb, 1024) for col in range(0, 1024, 128) for rb in (0, 1024)
)
NG = len(GROUPS)


def _link(gi, s):
    col_i, half = gi // 2, gi % 2
    return MASKS[(col_i + half + s) % 3]


def _aligned(x, m):
    return pl.multiple_of(x, m)


def kernel(t):
    m, n = t.shape
    assert (m, n) == (2048, 1024)
    bf16 = jnp.bfloat16

    def body(x_ref, out_ref, acc_ref, sb, c1, c2, c3,
             rs_send_sems, ag_send_sems, rs_sems, ag_sems):
        my = lax.axis_index("i")
        comms = [c1, c2, c3]
        bit0, bit1, bit2 = my & 1, (my >> 1) & 1, (my >> 2) & 1
        beta_of = {1: bit0 ^ bit1, 3: bit1, 4: bit2}

        barrier = pltpu.get_barrier_semaphore()
        for mask in MASKS:
            pl.semaphore_signal(
                barrier, inc=1,
                device_id=(my ^ mask,), device_id_type=pl.DeviceIdType.MESH,
            )
        pl.semaphore_wait(barrier, 3)

        offs = [jnp.int32(rb) for (_, _, rb, _) in GROUPS]

        def rs_send(gi, s):
            cg, w, rb, R = GROUPS[gi]
            mask = _link(gi, s)
            rows = (R // 2, R // 4, R // 4)[s]
            if s == 2:
                lo = offs[gi]
            else:
                lower = beta_of[mask] == 0
                lo = offs[gi] + jnp.where(lower, jnp.int32(rows), jnp.int32(0))
            lo = _aligned(lo, rows)
            src = x_ref if s == 0 else acc_ref
            sb[pl.ds(lo, rows), pl.ds(cg, w)] = src[
                pl.ds(lo, rows), pl.ds(cg, w)
            ].astype(bf16)
            rdma = pltpu.make_async_remote_copy(
                src_ref=sb.at[pl.ds(lo, rows), pl.ds(cg, w)],
                dst_ref=comms[s].at[pl.ds(lo, rows), pl.ds(cg, w)],
                send_sem=rs_send_sems.at[gi, s],
                recv_sem=rs_sems.at[gi, s],
                device_id=(my ^ mask,),
                device_id_type=pl.DeviceIdType.MESH,
            )
            rdma.start()
            return rdma

        def rs_accum(gi, s):
            cg, w, rb, R = GROUPS[gi]
            mask = _link(gi, s)
            half = (R // 2, R // 4)[s]
            lower = beta_of[mask] == 0
            keep = offs[gi] + jnp.where(lower, jnp.int32(0), jnp.int32(half))
            keep = _aligned(keep, half)
            src = x_ref if s == 0 else acc_ref
            acc_ref[pl.ds(keep, half), pl.ds(cg, w)] = (
                src[pl.ds(keep, half), pl.ds(cg, w)]
                + comms[s][pl.ds(keep, half), pl.ds(cg, w)].astype(jnp.float32)
            )
            offs[gi] = keep

        rs_d = [[None] * 3 for _ in range(NG)]
        for gi in range(NG):
            rs_d[gi][0] = rs_send(gi, 0)
        for s in (1, 2):
            for gi in range(NG):
                rs_d[gi][s - 1].wait_recv()
                rs_accum(gi, s - 1)
                rs_d[gi][s] = rs_send(gi, s)

        ag_recv = [[None] * 3 for _ in range(NG)]
        ag_poff = [[None] * 3 for _ in range(NG)]
        ag_sd = [[None] * 3 for _ in range(NG)]

        def ag_start(gi, a):
            cg, w, rb, R = GROUPS[gi]
            mask = _link(gi, 2 - a)
            lower = beta_of[mask] == 0
            sz = (R // 4) * (1 << (a - 1))
            o = _aligned(offs[gi], sz)
            send = pltpu.make_async_remote_copy(
                src_ref=out_ref.at[pl.ds(o, sz), pl.ds(cg, w)],
                dst_ref=out_ref.at[pl.ds(o, sz), pl.ds(cg, w)],
                send_sem=ag_send_sems.at[gi, a],
                recv_sem=ag_sems.at[gi, a],
                device_id=(my ^ mask,),
                device_id_type=pl.DeviceIdType.MESH,
            )
            send.start()
            ag_sd[gi][a] = send
            p_off = _aligned(jnp.where(lower, o + sz, o - sz), sz)
            recv = pltpu.make_async_remote_copy(
                src_ref=out_ref.at[pl.ds(p_off, sz), pl.ds(cg, w)],
                dst_ref=out_ref.at[pl.ds(p_off, sz), pl.ds(cg, w)],
                send_sem=ag_send_sems.at[gi, a],
                recv_sem=ag_sems.at[gi, a],
                device_id=(my ^ mask,),
                device_id_type=pl.DeviceIdType.MESH,
            )
            ag_recv[gi][a] = recv
            ag_poff[gi][a] = p_off

        for gi in range(NG):
            cg, w, rb, R = GROUPS[gi]
            q = R // 4
            rs_d[gi][2].wait_recv()
            o = _aligned(offs[gi], q)
            s_val = (
                acc_ref[pl.ds(o, q), pl.ds(cg, w)]
                + comms[2][pl.ds(o, q), pl.ds(cg, w)].astype(jnp.float32)
            )
            relu = jnp.maximum(s_val, 0.0)
            y = jnp.tanh(s_val) * s_val * s_val + relu * relu * relu
            out_ref[pl.ds(o, q), pl.ds(cg, w)] = y.astype(bf16)
            ag_start(gi, 1)

        for gi in range(NG):
            ag_recv[gi][1].wait_recv()
            offs[gi] = jnp.minimum(offs[gi], ag_poff[gi][1])
            ag_start(gi, 2)
        for gi in range(NG):
            ag_recv[gi][2].wait_recv()
        for gi in range(NG):
            for s in range(3):
                rs_d[gi][s].wait_send()
            for a in (1, 2):
                ag_sd[gi][a].wait_send()

    return pl.pallas_call(
        body,
        out_shape=jax.ShapeDtypeStruct((m, n), bf16),
        in_specs=[pl.BlockSpec(memory_space=pltpu.VMEM)],
        out_specs=pl.BlockSpec(memory_space=pltpu.VMEM),
        scratch_shapes=[
            pltpu.VMEM((m, n), jnp.float32),
            pltpu.VMEM((m, n), bf16),
            pltpu.VMEM((m, n), bf16),
            pltpu.VMEM((m, n), bf16),
            pltpu.VMEM((m, n), bf16),
            pltpu.SemaphoreType.DMA((NG, 3)),
            pltpu.SemaphoreType.DMA((NG, 3)),
            pltpu.SemaphoreType.DMA((NG, 3)),
            pltpu.SemaphoreType.DMA((NG, 3)),
        ],
        compiler_params=pltpu.CompilerParams(collective_id=0),
    )(t)
